# Initial kernel scaffold; baseline (speedup 1.0000x reference)
import jax
import jax.numpy as jnp
from jax.experimental import pallas as pl


def kernel(emb, W1, root1, b1, W2, root2, b2, edge_index, edge_type):
    raise NotImplementedError("write your pallas kernel here")


# speedup vs baseline: 11.1170x; 11.1170x over previous
import functools

import jax
import jax.numpy as jnp
from jax import lax
from jax.experimental import pallas as pl
from jax.experimental.pallas import tpu as pltpu
from jax.experimental.pallas import tpu_sc as plsc

NC = 2
NS = 16
L = 16
CH = 128

_MESH = plsc.VectorSubcoreMesh(
    core_axis_name="c", subcore_axis_name="s", num_cores=NC, num_subcores=NS)


def _i32(x):
  return jnp.asarray(x, dtype=jnp.int32)


def _make_pass_a(Ep, N, R):
  NW = NC * NS
  PW = Ep // NW
  KW = PW // CH
  ET = Ep // NS
  KT = ET // CH
  NR = N * R
  NRT = NR // NS

  def body(dst_h, rel_h, src_h, val_h, norm_h, gidx_h,
           b0, b1, b2, bsrc, bno, bidx, nbuf, cnt_sp):
    c = lax.axis_index("c")
    s = lax.axis_index("s")
    w = s * NC + c

    def z16(i, _):
      nbuf[pl.ds(i * L, L)] = jnp.zeros((L,), jnp.float32)
      return _
    lax.fori_loop(0, NRT // L, z16, None)
    pltpu.sync_copy(nbuf, cnt_sp.at[pl.ds(s * NRT, NRT)])
    plsc.subcore_barrier()

    tbase = s * ET
    pltpu.sync_copy(dst_h.at[pl.ds(tbase, ET)], b0)
    pltpu.sync_copy(rel_h.at[pl.ds(tbase, ET)], b1)
    pltpu.sync_copy(val_h.at[pl.ds(tbase, ET)], b2)

    def count_chunk(j, _):
      for k in range(CH // L):
        off = j * CH + k * L
        d = b0[pl.ds(off, L)]
        r = b1[pl.ds(off, L)]
        bidx[j, k * L:(k + 1) * L] = d * R + r
      pltpu.sync_copy(b2.at[pl.ds(j * CH, CH)],
                      cnt_sp.at[bidx.at[j]], add=True)
      return _
    lax.fori_loop(0, KT, count_chunk, None)
    plsc.subcore_barrier()

    pltpu.sync_copy(cnt_sp.at[pl.ds(s * NRT, NRT)], nbuf)

    def inv16(i, _):
      v = nbuf[pl.ds(i * L, L)]
      nbuf[pl.ds(i * L, L)] = 1.0 / jnp.maximum(v, 1.0)
      return _
    lax.fori_loop(0, NRT // L, inv16, None)
    pltpu.sync_copy(nbuf, cnt_sp.at[pl.ds(s * NRT, NRT)])
    plsc.subcore_barrier()

    wbase = w * PW
    pltpu.sync_copy(dst_h.at[pl.ds(wbase, PW)], b0.at[pl.ds(0, PW)])
    pltpu.sync_copy(rel_h.at[pl.ds(wbase, PW)], b1.at[pl.ds(0, PW)])
    pltpu.sync_copy(val_h.at[pl.ds(wbase, PW)], b2.at[pl.ds(0, PW)])
    pltpu.sync_copy(src_h.at[pl.ds(wbase, PW)], bsrc)

    def norm_chunk(j, _):
      for k in range(CH // L):
        off = j * CH + k * L
        d = b0[pl.ds(off, L)]
        r = b1[pl.ds(off, L)]
        bidx[j, k * L:(k + 1) * L] = d * R + r
        sv = bsrc[pl.ds(off, L)]
        bsrc[pl.ds(off, L)] = sv * R + r
      pltpu.sync_copy(cnt_sp.at[bidx.at[j]], bno.at[pl.ds(j * CH, CH)])
      for k in range(CH // L):
        off = j * CH + k * L
        bno[pl.ds(off, L)] = bno[pl.ds(off, L)] * b2[pl.ds(off, L)]
      return _
    lax.fori_loop(0, KW, norm_chunk, None)
    pltpu.sync_copy(bno, norm_h.at[pl.ds(wbase, PW)])
    pltpu.sync_copy(bsrc, gidx_h.at[pl.ds(wbase, PW)])

  return pl.kernel(
      body,
      out_type=(jax.ShapeDtypeStruct((Ep,), jnp.float32),
                jax.ShapeDtypeStruct((Ep,), jnp.int32)),
      mesh=_MESH,
      scratch_types=[
          pltpu.VMEM((ET,), jnp.int32),
          pltpu.VMEM((ET,), jnp.int32),
          pltpu.VMEM((ET,), jnp.float32),
          pltpu.VMEM((PW,), jnp.int32),
          pltpu.VMEM((PW,), jnp.float32),
          pltpu.VMEM((KT, CH), jnp.int32),
          pltpu.VMEM((NRT,), jnp.float32),
          pltpu.VMEM_SHARED((NR,), jnp.float32),
      ],
  )


def _make_pass_b(Ep, Np, D):
  NW = NC * NS
  PW = Ep // NW
  KW = PW // CH
  RT = Np // NS
  KR = RT // CH

  def body(h_h, gidx_h, norm_h, dst_h, out_h, bgi, bno, bdst, msg, agg_sp):
    c = lax.axis_index("c")
    s = lax.axis_index("s")
    w = s * NC + c

    for i in range(CH):
      for k in range(D // L):
        msg[i, k * L:(k + 1) * L] = jnp.zeros((L,), jnp.float32)
    for k in range(KR):
      pltpu.sync_copy(msg, agg_sp.at[pl.ds(s * RT + k * CH, CH)])
    plsc.subcore_barrier()

    wbase = w * PW
    pltpu.sync_copy(gidx_h.at[pl.ds(wbase, PW)], bgi)
    pltpu.sync_copy(norm_h.at[pl.ds(wbase, PW)], bno)
    pltpu.sync_copy(dst_h.at[w], bdst)

    def chunk(j, _):
      pltpu.sync_copy(h_h.at[bgi.at[pl.ds(j * CH, CH)]], msg)
      for g in range(CH // L):
        normv = bno[pl.ds(j * CH + g * L, L)]
        for t in range(L):
          i = g * L + t
          for k in range(D // L):
            msg[i, k * L:(k + 1) * L] = msg[i, k * L:(k + 1) * L] * normv[t]
      pltpu.sync_copy(msg, agg_sp.at[bdst.at[j]], add=True)
      return _
    lax.fori_loop(0, KW, chunk, None)
    plsc.subcore_barrier()

    for k in range(KR):
      rows = s * RT + k * CH
      pltpu.sync_copy(agg_sp.at[pl.ds(rows, CH)],
                      out_h.at[pl.ds(c * Np + rows, CH)])

  return pl.kernel(
      body,
      out_type=jax.ShapeDtypeStruct((NC * Np, D), jnp.float32),
      mesh=_MESH,
      scratch_types=[
          pltpu.VMEM((PW,), jnp.int32),
          pltpu.VMEM((PW,), jnp.float32),
          pltpu.VMEM((KW, CH), jnp.int32),
          pltpu.VMEM((CH, D), jnp.float32),
          pltpu.VMEM_SHARED((Np, D), jnp.float32),
      ],
  )


def _tc_matmul(x, w, bn=1024):
  Np, K = x.shape
  M = w.shape[1]

  def body(x_ref, w_ref, o_ref):
    o_ref[...] = jnp.dot(x_ref[...], w_ref[...],
                         preferred_element_type=jnp.float32)

  return pl.pallas_call(
      body,
      grid=(Np // bn,),
      in_specs=[pl.BlockSpec((bn, K), lambda i: (i, 0)),
                pl.BlockSpec((K, M), lambda i: (0, 0))],
      out_specs=pl.BlockSpec((bn, M), lambda i: (i, 0)),
      out_shape=jax.ShapeDtypeStruct((Np, M), jnp.float32),
  )(x, w)


def _tc_combine(aggflat, x, root, b, act, bn=1024):
  Np, K = x.shape
  D = root.shape[1]
  Da = aggflat.shape[1]
  nblk = Np // bn

  def body(a0_ref, a1_ref, x_ref, r_ref, b_ref, o_ref):
    acc = a0_ref[...][:, :D] + a1_ref[...][:, :D] + jnp.dot(
        x_ref[...], r_ref[...], preferred_element_type=jnp.float32)
    acc = acc + b_ref[...]
    if act == "relu":
      acc = jnp.maximum(acc, 0.0)
    elif act == "sigmoid":
      acc = jax.nn.sigmoid(acc)
    o_ref[...] = acc

  return pl.pallas_call(
      body,
      grid=(nblk,),
      in_specs=[pl.BlockSpec((bn, Da), lambda i: (i, 0)),
                pl.BlockSpec((bn, Da), lambda i: (i + nblk, 0)),
                pl.BlockSpec((bn, K), lambda i: (i, 0)),
                pl.BlockSpec((K, D), lambda i: (0, 0)),
                pl.BlockSpec((1, D), lambda i: (0, 0))],
      out_specs=pl.BlockSpec((bn, D), lambda i: (i, 0)),
      out_shape=jax.ShapeDtypeStruct((Np, D), jnp.float32),
  )(aggflat, aggflat, x, root, b[None, :])


def kernel(emb, W1, root1, b1, W2, root2, b2, edge_index, edge_type):
  N, Din = emb.shape
  R = W1.shape[0]
  Dh = W1.shape[2]
  Do = W2.shape[2]
  E = edge_type.shape[0]

  NW = NC * NS
  step = NW * CH
  Ep = ((E + step - 1) // step) * step
  Np = ((N + 2047) // 2048) * 2048

  src = edge_index[0]
  dst = edge_index[1]
  pad = Ep - E
  src_p = jnp.concatenate([src, jnp.zeros((pad,), jnp.int32)])
  dst_p = jnp.concatenate([dst, jnp.zeros((pad,), jnp.int32)])
  rel_p = jnp.concatenate([edge_type, jnp.zeros((pad,), jnp.int32)])
  val_p = jnp.concatenate([jnp.ones((E,), jnp.float32),
                           jnp.zeros((pad,), jnp.float32)])
  dst3d = dst_p.reshape(NW, Ep // NW // CH, CH)

  x_p = jnp.concatenate([emb, jnp.zeros((Np - N, Din), jnp.float32)])

  W1cat = jnp.transpose(W1, (1, 0, 2)).reshape(Din, R * Dh)
  Dp = 128
  W2pad = jnp.pad(W2, ((0, 0), (0, 0), (0, Dp - Do)))
  W2cat = jnp.transpose(W2pad, (1, 0, 2)).reshape(Dh, R * Dp)

  norm_e, gidx = _make_pass_a(Ep, N, R)(dst_p, rel_p, src_p, val_p)

  h1 = _tc_matmul(x_p, W1cat).reshape(Np * R, Dh)
  agg1 = _make_pass_b(Ep, Np, Dh)(h1, gidx, norm_e, dst3d)
  x1 = _tc_combine(agg1, x_p, root1, b1, "relu")

  h2 = _tc_matmul(x1, W2cat).reshape(Np * R, Dp)
  agg2 = _make_pass_b(Ep, Np, Dp)(h2, gidx, norm_e, dst3d)
  out = _tc_combine(agg2, x1, root2, b2, "sigmoid")

  return out[:N]
